# baseline (speedup 1.0000x reference)
import functools

import numpy as np

import jax
import jax.numpy as jnp
from jax import lax
from jax.experimental import pallas as pl
from jax.experimental.pallas import tpu as pltpu
from jax.experimental.pallas import tpu_sc as plsc

_M = 16384
_K = 10
_NC = 2
_NS = 16
_L = 16
_NW = _NC * _NS
_EPT = _M // _NS
_ECH = _EPT // _L
_QPW = (_K * _M) // _NW
_LN2 = 0.6931471805599453


def _np_threefry2x32(k0, k1, x0, x1):
    rot1 = (13, 15, 26, 6)
    rot2 = (17, 29, 16, 24)
    ks0 = np.uint32(k0)
    ks1 = np.uint32(k1)
    ks2 = np.uint32(0x1BD11BDA) ^ ks0 ^ ks1
    x0 = (x0 + ks0).astype(np.uint32)
    x1 = (x1 + ks1).astype(np.uint32)

    def rotl(v, d):
        return ((v << np.uint32(d)) | (v >> np.uint32(32 - d))).astype(np.uint32)

    def rounds(x0, x1, rots):
        for r in rots:
            x0 = (x0 + x1).astype(np.uint32)
            x1 = rotl(x1, r) ^ x0
        return x0, x1

    for i, rots in enumerate((rot1, rot2, rot1, rot2, rot1)):
        x0, x1 = rounds(x0, x1, rots)
        ks = (ks0, ks1, ks2)
        x0 = (x0 + ks[(i + 1) % 3]).astype(np.uint32)
        x1 = (x1 + ks[(i + 2) % 3] + np.uint32(i + 1)).astype(np.uint32)
    return x0, x1


def _np_uniform_key42(n):
    b0, b1 = _np_threefry2x32(0, 42, np.zeros(n, np.uint32),
                              np.arange(n, dtype=np.uint32))
    bits = b0 ^ b1
    f = ((bits >> np.uint32(9)) | np.uint32(0x3F800000)).view(np.float32)
    return f - np.float32(1.0)


_U = _np_uniform_key42(_K * _M)


def _bank_spread(i):
    return i ^ (((i >> 4) ^ (i >> 8) ^ (i >> 12)) & 15)


def _log1p_pos(x):
    y = 1.0 + x
    yi = plsc.bitcast(y, jnp.int32)
    e = ((yi >> 23) & 0xFF) - 127
    m = plsc.bitcast((yi & 0x7FFFFF) | 0x3F800000, jnp.float32)
    s = (m - 1.0) / (m + 1.0)
    s2 = s * s
    p = 1.0 + s2 * (1.0 / 3 + s2 * (1.0 / 5 + s2 * (1.0 / 7 + s2 * (1.0 / 9))))
    lnm = (2.0 * s) * p
    return e.astype(jnp.float32) * _LN2 + lnm


def _body(out_hbm, lab_hbm, u_hbm, part_hbm,
          elems_v, lab_v, idx_v, cumw_part_v, exch_i_v, exch_f_v, exch_f2_v,
          cumw_v, outall_v, posv_v, u_v, rowf_v, rowi_v, dma_sem,
          sh_cnt, sh_min, sh_sumneg, sh_pos, sh_cumw):
    c = lax.axis_index("c")
    s = lax.axis_index("s")
    wid = s * _NC + c
    lanes = lax.iota(jnp.int32, _L)

    base = pl.multiple_of(s * _EPT, _EPT)
    srow = pl.multiple_of(s * _L, _L)
    with jax.named_scope("stage_in"):
        outall_cp = pltpu.make_async_copy(out_hbm, outall_v, dma_sem)
        outall_cp.start()
        pltpu.sync_copy(out_hbm.at[pl.ds(base, _EPT)], elems_v)
        pltpu.sync_copy(lab_hbm.at[pl.ds(base, _EPT)], lab_v)

    def p1(i, carry):
        cnt, minv, snv = carry
        off = i * _L + lanes
        ov = plsc.load_gather(elems_v, [off])
        lv = plsc.load_gather(lab_v, [off])
        ispos = lv == 1
        cnt = cnt + jnp.max(plsc.all_reduce_population_count(ispos))
        minv = jnp.minimum(minv, jnp.where(ispos, jnp.inf, ov))
        snv = snv + jnp.where(ispos, jnp.float32(0.0), ov)
        return cnt, minv, snv

    minv0 = jnp.full((_L,), jnp.inf, dtype=jnp.float32)
    with jax.named_scope("pass1"):
        cnt, minv, snv = lax.fori_loop(
            0, _ECH, p1, (jnp.int32(0), minv0, jnp.zeros((_L,), jnp.float32)))
    min_s = jnp.min(minv)
    sumneg_s = jnp.sum(snv)

    rowi_v[...] = jnp.full((_L,), cnt, dtype=jnp.int32)
    pltpu.sync_copy(rowi_v, sh_cnt.at[pl.ds(srow, _L)])
    rowf_v[...] = jnp.full((_L,), min_s, dtype=jnp.float32)
    pltpu.sync_copy(rowf_v, sh_min.at[pl.ds(srow, _L)])
    rowf_v[...] = jnp.full((_L,), sumneg_s, dtype=jnp.float32)
    pltpu.sync_copy(rowf_v, sh_sumneg.at[pl.ds(srow, _L)])
    plsc.subcore_barrier()

    pltpu.sync_copy(sh_cnt, exch_i_v)
    pltpu.sync_copy(sh_min, exch_f_v)
    pltpu.sync_copy(sh_sumneg, exch_f2_v)
    gidx = lanes * _L
    c16 = plsc.load_gather(exch_i_v, [gidx])
    m16 = plsc.load_gather(exch_f_v, [gidx])
    sn16 = plsc.load_gather(exch_f2_v, [gidx])
    P = jnp.sum(c16)
    gmin = jnp.min(m16)
    excl = plsc.cumsum(c16) - c16
    pos_off = jnp.sum(jnp.where(lanes == s, excl, 0))
    w16 = sn16 - (_EPT - c16).astype(jnp.float32) * gmin
    W = jnp.sum(w16)
    wexcl = plsc.cumsum(w16) - w16
    w_off = jnp.sum(jnp.where(lanes == s, wexcl, jnp.float32(0.0)))

    def p2(i, carry):
        runc, cumc = carry
        off = i * _L + lanes
        ov = plsc.load_gather(elems_v, [off])
        lv = plsc.load_gather(lab_v, [off])
        ispos = lv == 1
        w = jnp.where(ispos, jnp.float32(0.0), ov - gmin)
        cw = plsc.cumsum(w) + cumc
        plsc.store_scatter(cumw_part_v, [_bank_spread(base + off) - base], cw)
        pm = ispos.astype(jnp.int32)
        ic = plsc.cumsum(pm)
        rank = pos_off + runc + (ic - pm)
        trash = _M + base + off
        plsc.store_scatter(idx_v, [off], jnp.where(ispos, rank, trash))
        return runc + jnp.max(ic), jnp.max(cw)

    with jax.named_scope("pass2"):
        _, _ = lax.fori_loop(0, _ECH, p2, (jnp.int32(0), w_off))

    with jax.named_scope("publish"):
        pltpu.sync_copy(cumw_part_v, sh_cumw.at[pl.ds(base, _EPT)])
        pltpu.sync_copy(elems_v, sh_pos.at[idx_v])
        plsc.subcore_barrier()

    with jax.named_scope("broadcast"):
        pltpu.sync_copy(sh_cumw, cumw_v)
        pltpu.sync_copy(sh_pos.at[pl.ds(0, _M)], posv_v)
        outall_cp.wait()

    Tq = _K * P
    q = ((Tq + (8 * _NW - 1)) // (8 * _NW)) * 8
    qstart = pl.multiple_of(wid * q, 8)
    count = jnp.clip(Tq - qstart, 0, q)
    nch = (count + (_L - 1)) // _L
    with jax.named_scope("u_copy"):
        pltpu.sync_copy(u_hbm.at[pl.ds(qstart, _QPW)], u_v)
    denom = Tq.astype(jnp.float32)

    sc_phase_b = jax.named_scope("phaseB")
    sc_phase_b.__enter__()

    @plsc.parallel_loop(0, nch, unroll=4, carry=jnp.zeros((_L,), jnp.float32))
    def acc(j, acc):
        loc = j * _L + lanes
        uv = plsc.load_gather(u_v, [loc])
        r = W * (1.0 - uv)

        def bstep(k, pos):
            step = _M // 2 >> k
            cv = plsc.load_gather(cumw_v, [_bank_spread(pos + (step - 1))])
            return pos + jnp.where(cv < r, step, 0)

        pos = lax.fori_loop(0, 14, bstep, jnp.zeros((_L,), jnp.int32))
        negv = plsc.load_gather(outall_v, [pos])
        pv = plsc.load_gather(posv_v, [(qstart + loc) % P])
        lny = _log1p_pos(jnp.exp(negv - pv))
        return acc + jnp.where(loc < count, lny, jnp.float32(0.0))

    sc_phase_b.__exit__(None, None, None)
    rowf_v[...] = acc / denom
    pltpu.sync_copy(rowf_v, part_hbm.at[wid])


@jax.jit
def _bpr_sc(output, label, u):
    mesh = plsc.VectorSubcoreMesh(core_axis_name="c", subcore_axis_name="s")
    f = functools.partial(
        pl.kernel,
        out_type=jax.ShapeDtypeStruct((_NW, _L), jnp.float32),
        mesh=mesh,
        compiler_params=pltpu.CompilerParams(needs_layout_passes=False),
        scratch_types=[
            pltpu.VMEM((_EPT,), jnp.float32),
            pltpu.VMEM((_EPT,), jnp.int32),
            pltpu.VMEM((_EPT,), jnp.int32),
            pltpu.VMEM((_EPT,), jnp.float32),
            pltpu.VMEM((_NS * _L,), jnp.int32),
            pltpu.VMEM((_NS * _L,), jnp.float32),
            pltpu.VMEM((_NS * _L,), jnp.float32),
            pltpu.VMEM((_M,), jnp.float32),
            pltpu.VMEM((_M,), jnp.float32),
            pltpu.VMEM((_M,), jnp.float32),
            pltpu.VMEM((_QPW,), jnp.float32),
            pltpu.VMEM((_L,), jnp.float32),
            pltpu.VMEM((_L,), jnp.int32),
            pltpu.SemaphoreType.DMA,
            pltpu.VMEM_SHARED((_NS * _L,), jnp.int32),
            pltpu.VMEM_SHARED((_NS * _L,), jnp.float32),
            pltpu.VMEM_SHARED((_NS * _L,), jnp.float32),
            pltpu.VMEM_SHARED((2 * _M,), jnp.float32),
            pltpu.VMEM_SHARED((_M,), jnp.float32),
        ],
    )(_body)
    return f(output, label, u)


def kernel(output, label):
    parts = _bpr_sc(output, label.astype(jnp.int32), jnp.asarray(_U))
    return jnp.sum(parts)


# speedup vs baseline: 1.3599x; 1.3599x over previous
import functools

import numpy as np

import jax
import jax.numpy as jnp
from jax import lax
from jax.experimental import pallas as pl
from jax.experimental.pallas import tpu as pltpu
from jax.experimental.pallas import tpu_sc as plsc

_M = 16384
_K = 10
_NC = 2
_NS = 16
_L = 16
_NW = _NC * _NS
_EPT = _M // _NS
_ECH = _EPT // _L
_QPW = (_K * _M) // _NW
_LN2 = 0.6931471805599453


def _np_threefry2x32(k0, k1, x0, x1):
    rot1 = (13, 15, 26, 6)
    rot2 = (17, 29, 16, 24)
    ks0 = np.uint32(k0)
    ks1 = np.uint32(k1)
    ks2 = np.uint32(0x1BD11BDA) ^ ks0 ^ ks1
    x0 = (x0 + ks0).astype(np.uint32)
    x1 = (x1 + ks1).astype(np.uint32)

    def rotl(v, d):
        return ((v << np.uint32(d)) | (v >> np.uint32(32 - d))).astype(np.uint32)

    def rounds(x0, x1, rots):
        for r in rots:
            x0 = (x0 + x1).astype(np.uint32)
            x1 = rotl(x1, r) ^ x0
        return x0, x1

    for i, rots in enumerate((rot1, rot2, rot1, rot2, rot1)):
        x0, x1 = rounds(x0, x1, rots)
        ks = (ks0, ks1, ks2)
        x0 = (x0 + ks[(i + 1) % 3]).astype(np.uint32)
        x1 = (x1 + ks[(i + 2) % 3] + np.uint32(i + 1)).astype(np.uint32)
    return x0, x1


def _np_uniform_key42(n):
    b0, b1 = _np_threefry2x32(0, 42, np.zeros(n, np.uint32),
                              np.arange(n, dtype=np.uint32))
    bits = b0 ^ b1
    f = ((bits >> np.uint32(9)) | np.uint32(0x3F800000)).view(np.float32)
    return f - np.float32(1.0)


_U = _np_uniform_key42(_K * _M)


def _bank_spread(i):
    return i ^ (((i >> 4) ^ (i >> 8) ^ (i >> 12)) & 15)


def _log1p_pos(x):
    y = 1.0 + x
    yi = plsc.bitcast(y, jnp.int32)
    e = ((yi >> 23) & 0xFF) - 127
    m = plsc.bitcast((yi & 0x7FFFFF) | 0x3F800000, jnp.float32)
    s = (m - 1.0) / (m + 1.0)
    s2 = s * s
    p = 1.0 + s2 * (1.0 / 3 + s2 * (1.0 / 5 + s2 * (1.0 / 7 + s2 * (1.0 / 9))))
    lnm = (2.0 * s) * p
    return e.astype(jnp.float32) * _LN2 + lnm


def _body(out_hbm, lab_hbm, u_hbm, part_hbm,
          elems_v, lab_v, idx_v, cumw_part_v, exch_i_v, exch_f_v, exch_f2_v,
          cumw_v, outall_v, posv_v, u_v, rowf_v, rowi_v, dma_sem,
          sh_cnt, sh_min, sh_sumneg, sh_pos, sh_cumw):
    c = lax.axis_index("c")
    s = lax.axis_index("s")
    wid = s * _NC + c
    lanes = lax.iota(jnp.int32, _L)

    base = pl.multiple_of(s * _EPT, _EPT)
    srow = pl.multiple_of(s * _L, _L)
    with jax.named_scope("stage_in"):
        outall_cp = pltpu.make_async_copy(out_hbm, outall_v, dma_sem)
        outall_cp.start()
        pltpu.sync_copy(out_hbm.at[pl.ds(base, _EPT)], elems_v)
        pltpu.sync_copy(lab_hbm.at[pl.ds(base, _EPT)], lab_v)

    def p1(i, carry):
        cnt, minv, snv = carry
        off = i * _L + lanes
        ov = plsc.load_gather(elems_v, [off])
        lv = plsc.load_gather(lab_v, [off])
        ispos = lv == 1
        cnt = cnt + jnp.max(plsc.all_reduce_population_count(ispos))
        minv = jnp.minimum(minv, jnp.where(ispos, jnp.inf, ov))
        snv = snv + jnp.where(ispos, jnp.float32(0.0), ov)
        return cnt, minv, snv

    minv0 = jnp.full((_L,), jnp.inf, dtype=jnp.float32)
    with jax.named_scope("pass1"):
        cnt, minv, snv = lax.fori_loop(
            0, _ECH, p1, (jnp.int32(0), minv0, jnp.zeros((_L,), jnp.float32)))
    min_s = jnp.min(minv)
    sumneg_s = jnp.sum(snv)

    rowi_v[...] = jnp.full((_L,), cnt, dtype=jnp.int32)
    pltpu.sync_copy(rowi_v, sh_cnt.at[pl.ds(srow, _L)])
    rowf_v[...] = jnp.full((_L,), min_s, dtype=jnp.float32)
    pltpu.sync_copy(rowf_v, sh_min.at[pl.ds(srow, _L)])
    rowf_v[...] = jnp.full((_L,), sumneg_s, dtype=jnp.float32)
    pltpu.sync_copy(rowf_v, sh_sumneg.at[pl.ds(srow, _L)])
    plsc.subcore_barrier()

    pltpu.sync_copy(sh_cnt, exch_i_v)
    pltpu.sync_copy(sh_min, exch_f_v)
    pltpu.sync_copy(sh_sumneg, exch_f2_v)
    gidx = lanes * _L
    c16 = plsc.load_gather(exch_i_v, [gidx])
    m16 = plsc.load_gather(exch_f_v, [gidx])
    sn16 = plsc.load_gather(exch_f2_v, [gidx])
    P = jnp.sum(c16)
    gmin = jnp.min(m16)
    excl = plsc.cumsum(c16) - c16
    pos_off = jnp.sum(jnp.where(lanes == s, excl, 0))
    w16 = sn16 - (_EPT - c16).astype(jnp.float32) * gmin
    W = jnp.sum(w16)
    wexcl = plsc.cumsum(w16) - w16
    w_off = jnp.sum(jnp.where(lanes == s, wexcl, jnp.float32(0.0)))

    def p2(i, carry):
        runc, cumc = carry
        off = i * _L + lanes
        ov = plsc.load_gather(elems_v, [off])
        lv = plsc.load_gather(lab_v, [off])
        ispos = lv == 1
        w = jnp.where(ispos, jnp.float32(0.0), ov - gmin)
        cw = plsc.cumsum(w) + cumc
        plsc.store_scatter(cumw_part_v, [_bank_spread(base + off) - base], cw)
        pm = ispos.astype(jnp.int32)
        ic = plsc.cumsum(pm)
        rank = pos_off + runc + (ic - pm)
        trash = _M + base + off
        plsc.store_scatter(idx_v, [off], jnp.where(ispos, rank, trash))
        return runc + jnp.max(ic), jnp.max(cw)

    with jax.named_scope("pass2"):
        _, _ = lax.fori_loop(0, _ECH, p2, (jnp.int32(0), w_off))

    with jax.named_scope("publish"):
        pltpu.sync_copy(cumw_part_v, sh_cumw.at[pl.ds(base, _EPT)])
        pltpu.sync_copy(elems_v, sh_pos.at[idx_v])
        plsc.subcore_barrier()

    with jax.named_scope("broadcast"):
        pltpu.sync_copy(sh_cumw, cumw_v)
        pltpu.sync_copy(sh_pos.at[pl.ds(0, _M)], posv_v)
        outall_cp.wait()

    Tq = _K * P
    q = ((Tq + (8 * _NW - 1)) // (8 * _NW)) * 8
    qstart = pl.multiple_of(wid * q, 8)
    count = jnp.clip(Tq - qstart, 0, q)
    nch = (count + (_L - 1)) // _L
    with jax.named_scope("u_copy"):
        pltpu.sync_copy(u_hbm.at[pl.ds(qstart, _QPW)], u_v)
    denom = Tq.astype(jnp.float32)

    sc_phase_b = jax.named_scope("phaseB")
    sc_phase_b.__enter__()

    @plsc.parallel_loop(0, nch, unroll=2, carry=jnp.zeros((_L,), jnp.float32))
    def acc(j, acc):
        loc = j * _L + lanes
        uv = plsc.load_gather(u_v, [loc])
        r = W * (1.0 - uv)
        pos = jnp.zeros((_L,), jnp.int32)
        step = _M // 2
        while step >= 1:
            cv = plsc.load_gather(cumw_v, [_bank_spread(pos + (step - 1))])
            pos = pos + jnp.where(cv < r, step, 0)
            step //= 2
        negv = plsc.load_gather(outall_v, [pos])
        pv = plsc.load_gather(posv_v, [(qstart + loc) % P])
        lny = _log1p_pos(jnp.exp(negv - pv))
        return acc + jnp.where(loc < count, lny, jnp.float32(0.0))

    sc_phase_b.__exit__(None, None, None)
    rowf_v[...] = acc / denom
    pltpu.sync_copy(rowf_v, part_hbm.at[wid])


@jax.jit
def _bpr_sc(output, label, u):
    mesh = plsc.VectorSubcoreMesh(core_axis_name="c", subcore_axis_name="s")
    f = functools.partial(
        pl.kernel,
        out_type=jax.ShapeDtypeStruct((_NW, _L), jnp.float32),
        mesh=mesh,
        compiler_params=pltpu.CompilerParams(needs_layout_passes=False),
        scratch_types=[
            pltpu.VMEM((_EPT,), jnp.float32),
            pltpu.VMEM((_EPT,), jnp.int32),
            pltpu.VMEM((_EPT,), jnp.int32),
            pltpu.VMEM((_EPT,), jnp.float32),
            pltpu.VMEM((_NS * _L,), jnp.int32),
            pltpu.VMEM((_NS * _L,), jnp.float32),
            pltpu.VMEM((_NS * _L,), jnp.float32),
            pltpu.VMEM((_M,), jnp.float32),
            pltpu.VMEM((_M,), jnp.float32),
            pltpu.VMEM((_M,), jnp.float32),
            pltpu.VMEM((_QPW,), jnp.float32),
            pltpu.VMEM((_L,), jnp.float32),
            pltpu.VMEM((_L,), jnp.int32),
            pltpu.SemaphoreType.DMA,
            pltpu.VMEM_SHARED((_NS * _L,), jnp.int32),
            pltpu.VMEM_SHARED((_NS * _L,), jnp.float32),
            pltpu.VMEM_SHARED((_NS * _L,), jnp.float32),
            pltpu.VMEM_SHARED((2 * _M,), jnp.float32),
            pltpu.VMEM_SHARED((_M,), jnp.float32),
        ],
    )(_body)
    return f(output, label, u)


def kernel(output, label):
    parts = _bpr_sc(output, label.astype(jnp.int32), jnp.asarray(_U))
    return jnp.sum(parts)


# speedup vs baseline: 1.5156x; 1.1145x over previous
import functools

import numpy as np

import jax
import jax.numpy as jnp
from jax import lax
from jax.experimental import pallas as pl
from jax.experimental.pallas import tpu as pltpu
from jax.experimental.pallas import tpu_sc as plsc

_M = 16384
_K = 10
_NC = 2
_NS = 16
_L = 16
_NW = _NC * _NS
_EPT = _M // _NS
_ECH = _EPT // _L
_QPW = (_K * _M) // _NW
_LN2 = 0.6931471805599453


def _np_threefry2x32(k0, k1, x0, x1):
    rot1 = (13, 15, 26, 6)
    rot2 = (17, 29, 16, 24)
    ks0 = np.uint32(k0)
    ks1 = np.uint32(k1)
    ks2 = np.uint32(0x1BD11BDA) ^ ks0 ^ ks1
    x0 = (x0 + ks0).astype(np.uint32)
    x1 = (x1 + ks1).astype(np.uint32)

    def rotl(v, d):
        return ((v << np.uint32(d)) | (v >> np.uint32(32 - d))).astype(np.uint32)

    def rounds(x0, x1, rots):
        for r in rots:
            x0 = (x0 + x1).astype(np.uint32)
            x1 = rotl(x1, r) ^ x0
        return x0, x1

    for i, rots in enumerate((rot1, rot2, rot1, rot2, rot1)):
        x0, x1 = rounds(x0, x1, rots)
        ks = (ks0, ks1, ks2)
        x0 = (x0 + ks[(i + 1) % 3]).astype(np.uint32)
        x1 = (x1 + ks[(i + 2) % 3] + np.uint32(i + 1)).astype(np.uint32)
    return x0, x1


def _np_uniform_key42(n):
    b0, b1 = _np_threefry2x32(0, 42, np.zeros(n, np.uint32),
                              np.arange(n, dtype=np.uint32))
    bits = b0 ^ b1
    f = ((bits >> np.uint32(9)) | np.uint32(0x3F800000)).view(np.float32)
    return f - np.float32(1.0)


_U = _np_uniform_key42(_K * _M)


def _bank_spread(i):
    return i ^ (((i >> 4) ^ (i >> 8) ^ (i >> 12)) & 15)


def _log1p_pos(x):
    y = 1.0 + x
    yi = plsc.bitcast(y, jnp.int32)
    e = ((yi >> 23) & 0xFF) - 127
    m = plsc.bitcast((yi & 0x7FFFFF) | 0x3F800000, jnp.float32)
    s = (m - 1.0) / (m + 1.0)
    s2 = s * s
    p = 1.0 + s2 * (1.0 / 3 + s2 * (1.0 / 5 + s2 * (1.0 / 7 + s2 * (1.0 / 9))))
    lnm = (2.0 * s) * p
    return e.astype(jnp.float32) * _LN2 + lnm


def _body(out_hbm, lab_hbm, u_hbm, part_hbm,
          elems_v, lab_v, idx_v, cumw_part_v, exch_i_v, exch_f_v, exch_f2_v,
          cumw_v, outall_v, posv_v, u_v, rowf_v, rowi_v, dma_sem,
          sh_cnt, sh_min, sh_sumneg, sh_pos, sh_cumw):
    c = lax.axis_index("c")
    s = lax.axis_index("s")
    wid = s * _NC + c
    lanes = lax.iota(jnp.int32, _L)

    base = pl.multiple_of(s * _EPT, _EPT)
    srow = pl.multiple_of(s * _L, _L)
    with jax.named_scope("stage_in"):
        outall_cp = pltpu.make_async_copy(out_hbm, outall_v, dma_sem)
        outall_cp.start()
        pltpu.sync_copy(out_hbm.at[pl.ds(base, _EPT)], elems_v)
        pltpu.sync_copy(lab_hbm.at[pl.ds(base, _EPT)], lab_v)

    def p1(i, carry):
        cnt, minv, snv = carry
        off = i * _L + lanes
        ov = plsc.load_gather(elems_v, [off])
        lv = plsc.load_gather(lab_v, [off])
        ispos = lv == 1
        cnt = cnt + jnp.max(plsc.all_reduce_population_count(ispos))
        minv = jnp.minimum(minv, jnp.where(ispos, jnp.inf, ov))
        snv = snv + jnp.where(ispos, jnp.float32(0.0), ov)
        return cnt, minv, snv

    minv0 = jnp.full((_L,), jnp.inf, dtype=jnp.float32)
    with jax.named_scope("pass1"):
        cnt, minv, snv = lax.fori_loop(
            0, _ECH, p1, (jnp.int32(0), minv0, jnp.zeros((_L,), jnp.float32)))
    min_s = jnp.min(minv)
    sumneg_s = jnp.sum(snv)

    rowi_v[...] = jnp.full((_L,), cnt, dtype=jnp.int32)
    pltpu.sync_copy(rowi_v, sh_cnt.at[pl.ds(srow, _L)])
    rowf_v[...] = jnp.full((_L,), min_s, dtype=jnp.float32)
    pltpu.sync_copy(rowf_v, sh_min.at[pl.ds(srow, _L)])
    rowf_v[...] = jnp.full((_L,), sumneg_s, dtype=jnp.float32)
    pltpu.sync_copy(rowf_v, sh_sumneg.at[pl.ds(srow, _L)])
    plsc.subcore_barrier()

    pltpu.sync_copy(sh_cnt, exch_i_v)
    pltpu.sync_copy(sh_min, exch_f_v)
    pltpu.sync_copy(sh_sumneg, exch_f2_v)
    gidx = lanes * _L
    c16 = plsc.load_gather(exch_i_v, [gidx])
    m16 = plsc.load_gather(exch_f_v, [gidx])
    sn16 = plsc.load_gather(exch_f2_v, [gidx])
    P = jnp.sum(c16)
    gmin = jnp.min(m16)
    excl = plsc.cumsum(c16) - c16
    pos_off = jnp.sum(jnp.where(lanes == s, excl, 0))
    w16 = sn16 - (_EPT - c16).astype(jnp.float32) * gmin
    W = jnp.sum(w16)
    wexcl = plsc.cumsum(w16) - w16
    w_off = jnp.sum(jnp.where(lanes == s, wexcl, jnp.float32(0.0)))

    def p2(i, carry):
        runc, cumc = carry
        off = i * _L + lanes
        ov = plsc.load_gather(elems_v, [off])
        lv = plsc.load_gather(lab_v, [off])
        ispos = lv == 1
        w = jnp.where(ispos, jnp.float32(0.0), ov - gmin)
        cw = plsc.cumsum(w) + cumc
        plsc.store_scatter(cumw_part_v, [_bank_spread(base + off) - base], cw)
        pm = ispos.astype(jnp.int32)
        ic = plsc.cumsum(pm)
        rank = pos_off + runc + (ic - pm)
        trash = _M + base + off
        plsc.store_scatter(idx_v, [off], jnp.where(ispos, rank, trash))
        return runc + jnp.max(ic), jnp.max(cw)

    with jax.named_scope("pass2"):
        _, _ = lax.fori_loop(0, _ECH, p2, (jnp.int32(0), w_off))

    with jax.named_scope("publish"):
        pltpu.sync_copy(cumw_part_v, sh_cumw.at[pl.ds(base, _EPT)])
        pltpu.sync_copy(elems_v, sh_pos.at[idx_v])
        plsc.subcore_barrier()

    with jax.named_scope("broadcast"):
        pltpu.sync_copy(sh_cumw, cumw_v)
        pltpu.sync_copy(sh_pos.at[pl.ds(0, _M)], posv_v)
        outall_cp.wait()

    Tq = _K * P
    q = ((Tq + (8 * _NW - 1)) // (8 * _NW)) * 8
    qstart = pl.multiple_of(wid * q, 8)
    count = jnp.clip(Tq - qstart, 0, q)
    nch = (count + (_L - 1)) // _L
    with jax.named_scope("u_copy"):
        pltpu.sync_copy(u_hbm.at[pl.ds(qstart, _QPW)], u_v)
    denom = Tq.astype(jnp.float32)

    sc_phase_b = jax.named_scope("phaseB")
    sc_phase_b.__enter__()

    @plsc.parallel_loop(0, nch, unroll=8, carry=jnp.zeros((_L,), jnp.float32))
    def acc(j, acc):
        loc = j * _L + lanes
        uv = plsc.load_gather(u_v, [loc])
        r = W * (1.0 - uv)
        pos = jnp.zeros((_L,), jnp.int32)
        step = _M // 2
        while step >= 1:
            cv = plsc.load_gather(cumw_v, [_bank_spread(pos + (step - 1))])
            pos = pos + jnp.where(cv < r, step, 0)
            step //= 2
        negv = plsc.load_gather(outall_v, [pos])
        pv = plsc.load_gather(posv_v, [(qstart + loc) % P])
        lny = _log1p_pos(jnp.exp(negv - pv))
        return acc + jnp.where(loc < count, lny, jnp.float32(0.0))

    sc_phase_b.__exit__(None, None, None)
    rowf_v[...] = acc / denom
    pltpu.sync_copy(rowf_v, part_hbm.at[wid])


@jax.jit
def _bpr_sc(output, label, u):
    mesh = plsc.VectorSubcoreMesh(core_axis_name="c", subcore_axis_name="s")
    f = functools.partial(
        pl.kernel,
        out_type=jax.ShapeDtypeStruct((_NW, _L), jnp.float32),
        mesh=mesh,
        compiler_params=pltpu.CompilerParams(needs_layout_passes=False),
        scratch_types=[
            pltpu.VMEM((_EPT,), jnp.float32),
            pltpu.VMEM((_EPT,), jnp.int32),
            pltpu.VMEM((_EPT,), jnp.int32),
            pltpu.VMEM((_EPT,), jnp.float32),
            pltpu.VMEM((_NS * _L,), jnp.int32),
            pltpu.VMEM((_NS * _L,), jnp.float32),
            pltpu.VMEM((_NS * _L,), jnp.float32),
            pltpu.VMEM((_M,), jnp.float32),
            pltpu.VMEM((_M,), jnp.float32),
            pltpu.VMEM((_M,), jnp.float32),
            pltpu.VMEM((_QPW,), jnp.float32),
            pltpu.VMEM((_L,), jnp.float32),
            pltpu.VMEM((_L,), jnp.int32),
            pltpu.SemaphoreType.DMA,
            pltpu.VMEM_SHARED((_NS * _L,), jnp.int32),
            pltpu.VMEM_SHARED((_NS * _L,), jnp.float32),
            pltpu.VMEM_SHARED((_NS * _L,), jnp.float32),
            pltpu.VMEM_SHARED((2 * _M,), jnp.float32),
            pltpu.VMEM_SHARED((_M,), jnp.float32),
        ],
    )(_body)
    return f(output, label, u)


def kernel(output, label):
    parts = _bpr_sc(output, label.astype(jnp.int32), jnp.asarray(_U))
    return jnp.sum(parts)


# speedup vs baseline: 1.5761x; 1.0399x over previous
import functools

import numpy as np

import jax
import jax.numpy as jnp
from jax import lax
from jax.experimental import pallas as pl
from jax.experimental.pallas import tpu as pltpu
from jax.experimental.pallas import tpu_sc as plsc

_M = 16384
_K = 10
_NC = 2
_NS = 16
_L = 16
_NW = _NC * _NS
_EPT = _M // _NS
_ECH = _EPT // _L
_QPW = (_K * _M) // _NW
_LN2 = 0.6931471805599453


def _np_threefry2x32(k0, k1, x0, x1):
    rot1 = (13, 15, 26, 6)
    rot2 = (17, 29, 16, 24)
    ks0 = np.uint32(k0)
    ks1 = np.uint32(k1)
    ks2 = np.uint32(0x1BD11BDA) ^ ks0 ^ ks1
    x0 = (x0 + ks0).astype(np.uint32)
    x1 = (x1 + ks1).astype(np.uint32)

    def rotl(v, d):
        return ((v << np.uint32(d)) | (v >> np.uint32(32 - d))).astype(np.uint32)

    def rounds(x0, x1, rots):
        for r in rots:
            x0 = (x0 + x1).astype(np.uint32)
            x1 = rotl(x1, r) ^ x0
        return x0, x1

    for i, rots in enumerate((rot1, rot2, rot1, rot2, rot1)):
        x0, x1 = rounds(x0, x1, rots)
        ks = (ks0, ks1, ks2)
        x0 = (x0 + ks[(i + 1) % 3]).astype(np.uint32)
        x1 = (x1 + ks[(i + 2) % 3] + np.uint32(i + 1)).astype(np.uint32)
    return x0, x1


def _np_uniform_key42(n):
    b0, b1 = _np_threefry2x32(0, 42, np.zeros(n, np.uint32),
                              np.arange(n, dtype=np.uint32))
    bits = b0 ^ b1
    f = ((bits >> np.uint32(9)) | np.uint32(0x3F800000)).view(np.float32)
    return f - np.float32(1.0)


_U = _np_uniform_key42(_K * _M)


def _bank_spread(i):
    return i ^ (((i >> 4) ^ (i >> 8) ^ (i >> 12)) & 15)


def _log1p_pos(x):
    y = 1.0 + x
    yi = plsc.bitcast(y, jnp.int32)
    e = ((yi >> 23) & 0xFF) - 127
    m = plsc.bitcast((yi & 0x7FFFFF) | 0x3F800000, jnp.float32)
    s = (m - 1.0) / (m + 1.0)
    s2 = s * s
    p = 1.0 + s2 * (1.0 / 3 + s2 * (1.0 / 5 + s2 * (1.0 / 7 + s2 * (1.0 / 9))))
    lnm = (2.0 * s) * p
    return e.astype(jnp.float32) * _LN2 + lnm


def _body(out_hbm, lab_hbm, u_hbm, part_hbm,
          elems_v, lab_v, idx_v, cumw_part_v, exch_i_v, exch_f_v, exch_f2_v,
          cumw_v, outall_v, posv_v, u_v, rowf_v, rowi_v,
          sem_a, sem_b, sem_c, sem_d,
          sh_cnt, sh_min, sh_sumneg, sh_pos, sh_cumw):
    c = lax.axis_index("c")
    s = lax.axis_index("s")
    wid = s * _NC + c
    lanes = lax.iota(jnp.int32, _L)

    base = pl.multiple_of(s * _EPT, _EPT)
    srow = pl.multiple_of(s * _L, _L)
    outall_cp = pltpu.make_async_copy(out_hbm, outall_v, sem_a)
    outall_cp.start()
    elems_cp = pltpu.make_async_copy(
        out_hbm.at[pl.ds(base, _EPT)], elems_v, sem_b)
    elems_cp.start()
    lab_cp = pltpu.make_async_copy(lab_hbm.at[pl.ds(base, _EPT)], lab_v, sem_c)
    lab_cp.start()
    elems_cp.wait()
    lab_cp.wait()

    def p1(i, carry):
        cnt, minv, snv = carry
        off = i * _L + lanes
        ov = plsc.load_gather(elems_v, [off])
        lv = plsc.load_gather(lab_v, [off])
        ispos = lv == 1
        cnt = cnt + jnp.max(plsc.all_reduce_population_count(ispos))
        minv = jnp.minimum(minv, jnp.where(ispos, jnp.inf, ov))
        snv = snv + jnp.where(ispos, jnp.float32(0.0), ov)
        return cnt, minv, snv

    minv0 = jnp.full((_L,), jnp.inf, dtype=jnp.float32)
    cnt, minv, snv = lax.fori_loop(
        0, _ECH, p1, (jnp.int32(0), minv0, jnp.zeros((_L,), jnp.float32)))
    min_s = jnp.min(minv)
    sumneg_s = jnp.sum(snv)

    rowi_v[...] = jnp.full((_L,), cnt, dtype=jnp.int32)
    pltpu.sync_copy(rowi_v, sh_cnt.at[pl.ds(srow, _L)])
    rowf_v[...] = jnp.full((_L,), min_s, dtype=jnp.float32)
    pltpu.sync_copy(rowf_v, sh_min.at[pl.ds(srow, _L)])
    rowf_v[...] = jnp.full((_L,), sumneg_s, dtype=jnp.float32)
    pltpu.sync_copy(rowf_v, sh_sumneg.at[pl.ds(srow, _L)])
    plsc.subcore_barrier()

    pltpu.sync_copy(sh_cnt, exch_i_v)
    pltpu.sync_copy(sh_min, exch_f_v)
    pltpu.sync_copy(sh_sumneg, exch_f2_v)
    gidx = lanes * _L
    c16 = plsc.load_gather(exch_i_v, [gidx])
    m16 = plsc.load_gather(exch_f_v, [gidx])
    sn16 = plsc.load_gather(exch_f2_v, [gidx])
    P = jnp.sum(c16)
    gmin = jnp.min(m16)
    excl = plsc.cumsum(c16) - c16
    pos_off = jnp.sum(jnp.where(lanes == s, excl, 0))
    w16 = sn16 - (_EPT - c16).astype(jnp.float32) * gmin
    W = jnp.sum(w16)
    wexcl = plsc.cumsum(w16) - w16
    w_off = jnp.sum(jnp.where(lanes == s, wexcl, jnp.float32(0.0)))

    Tq = _K * P
    q = ((Tq + (8 * _NW - 1)) // (8 * _NW)) * 8
    qstart = pl.multiple_of(wid * q, 8)
    count = jnp.clip(Tq - qstart, 0, q)
    nch = (count + (_L - 1)) // _L
    u_cp = pltpu.make_async_copy(u_hbm.at[pl.ds(qstart, _QPW)], u_v, sem_b)
    u_cp.start()

    def p2(i, carry):
        runc, cumc = carry
        off = i * _L + lanes
        ov = plsc.load_gather(elems_v, [off])
        lv = plsc.load_gather(lab_v, [off])
        ispos = lv == 1
        w = jnp.where(ispos, jnp.float32(0.0), ov - gmin)
        cw = plsc.cumsum(w) + cumc
        plsc.store_scatter(cumw_part_v, [_bank_spread(base + off) - base], cw)
        pm = ispos.astype(jnp.int32)
        ic = plsc.cumsum(pm)
        rank = pos_off + runc + (ic - pm)
        trash = _M + base + off
        plsc.store_scatter(idx_v, [off], jnp.where(ispos, rank, trash))
        return runc + jnp.max(ic), jnp.max(cw)

    _, _ = lax.fori_loop(0, _ECH, p2, (jnp.int32(0), w_off))

    pltpu.sync_copy(cumw_part_v, sh_cumw.at[pl.ds(base, _EPT)])
    pltpu.sync_copy(elems_v, sh_pos.at[idx_v])
    plsc.subcore_barrier()

    cumw_cp = pltpu.make_async_copy(sh_cumw, cumw_v, sem_c)
    cumw_cp.start()
    posv_cp = pltpu.make_async_copy(sh_pos.at[pl.ds(0, _M)], posv_v, sem_d)
    posv_cp.start()
    outall_cp.wait()
    cumw_cp.wait()
    posv_cp.wait()
    u_cp.wait()

    denom = Tq.astype(jnp.float32)

    @plsc.parallel_loop(0, nch, unroll=4, carry=jnp.zeros((_L,), jnp.float32))
    def acc(j, acc):
        loc = j * _L + lanes
        uv = plsc.load_gather(u_v, [loc])
        r = W * (1.0 - uv)
        pos = jnp.zeros((_L,), jnp.int32)
        step = _M // 2
        while step >= 1:
            cv = plsc.load_gather(cumw_v, [_bank_spread(pos + (step - 1))])
            pos = pos + jnp.where(cv < r, step, 0)
            step //= 2
        negv = plsc.load_gather(outall_v, [pos])
        pv = plsc.load_gather(posv_v, [(qstart + loc) % P])
        lny = _log1p_pos(jnp.exp(negv - pv))
        return acc + jnp.where(loc < count, lny, jnp.float32(0.0))

    rowf_v[...] = acc / denom
    pltpu.sync_copy(rowf_v, part_hbm.at[wid])


@jax.jit
def _bpr_sc(output, label, u):
    mesh = plsc.VectorSubcoreMesh(core_axis_name="c", subcore_axis_name="s")
    f = functools.partial(
        pl.kernel,
        out_type=jax.ShapeDtypeStruct((_NW, _L), jnp.float32),
        mesh=mesh,
        compiler_params=pltpu.CompilerParams(needs_layout_passes=False),
        scratch_types=[
            pltpu.VMEM((_EPT,), jnp.float32),
            pltpu.VMEM((_EPT,), jnp.int32),
            pltpu.VMEM((_EPT,), jnp.int32),
            pltpu.VMEM((_EPT,), jnp.float32),
            pltpu.VMEM((_NS * _L,), jnp.int32),
            pltpu.VMEM((_NS * _L,), jnp.float32),
            pltpu.VMEM((_NS * _L,), jnp.float32),
            pltpu.VMEM((_M,), jnp.float32),
            pltpu.VMEM((_M,), jnp.float32),
            pltpu.VMEM((_M,), jnp.float32),
            pltpu.VMEM((_QPW,), jnp.float32),
            pltpu.VMEM((_L,), jnp.float32),
            pltpu.VMEM((_L,), jnp.int32),
            pltpu.SemaphoreType.DMA,
            pltpu.SemaphoreType.DMA,
            pltpu.SemaphoreType.DMA,
            pltpu.SemaphoreType.DMA,
            pltpu.VMEM_SHARED((_NS * _L,), jnp.int32),
            pltpu.VMEM_SHARED((_NS * _L,), jnp.float32),
            pltpu.VMEM_SHARED((_NS * _L,), jnp.float32),
            pltpu.VMEM_SHARED((2 * _M,), jnp.float32),
            pltpu.VMEM_SHARED((_M,), jnp.float32),
        ],
    )(_body)
    return f(output, label, u)


def kernel(output, label):
    parts = _bpr_sc(output, label.astype(jnp.int32), jnp.asarray(_U))
    return jnp.sum(parts)
